# Initial kernel scaffold; baseline (speedup 1.0000x reference)
import jax
import jax.numpy as jnp
from jax.experimental import pallas as pl


def kernel(x, weights):
    raise NotImplementedError("write your pallas kernel here")


# speedup vs baseline: 1.5120x; 1.5120x over previous
import jax
import jax.numpy as jnp
from jax.experimental import pallas as pl
from jax.experimental.pallas import tpu as pltpu

_C = 96
_EPS = 1e-06
_NORM = float(_C)
_LANES = 512


def _softmax_kernel(w_ref, wt_ref):
    w = w_ref[...]
    m = jnp.max(w)
    e = jnp.exp(w - m)
    wt_ref[...] = (e / jnp.sum(e)) * _NORM


def _rank_weight_kernel(wt_ref, x_ref, o_ref):
    x = x_ref[0]
    c_iota = jax.lax.broadcasted_iota(jnp.int32, (_C, 1), 0)

    def rank_body(j, acc):
        row = x_ref[0, pl.ds(j, 1), :]
        gt = row > x
        ge = row >= x
        beat = gt | (ge & (c_iota > j))
        return acc + jnp.where(beat, jnp.int32(1), jnp.int32(0))

    rank = jax.lax.fori_loop(0, _C, rank_body, jnp.zeros(x.shape, jnp.int32))

    def sel_body(j, acc):
        w = wt_ref[j]
        return jnp.where(rank == j, w, acc)

    wt_g = jax.lax.fori_loop(0, _C, sel_body, jnp.zeros(x.shape, jnp.float32))
    o_ref[0] = wt_g * jax.lax.rsqrt(x * x + _EPS)


@jax.jit
def kernel(x, weights):
    b, c, h, w = x.shape
    wt = pl.pallas_call(
        _softmax_kernel,
        out_shape=jax.ShapeDtypeStruct((1, _C), jnp.float32),
    )(weights.reshape(1, _C))
    wt = wt.reshape(_C)

    xs = x.reshape(b, c, h * w)
    n_s = (h * w) // _LANES
    out = pl.pallas_call(
        _rank_weight_kernel,
        grid=(b, n_s),
        in_specs=[
            pl.BlockSpec(memory_space=pltpu.SMEM),
            pl.BlockSpec((1, c, _LANES), lambda i, j: (i, 0, j)),
        ],
        out_specs=pl.BlockSpec((1, c, _LANES), lambda i, j: (i, 0, j)),
        out_shape=jax.ShapeDtypeStruct((b, c, h * w), jnp.float32),
        compiler_params=pltpu.CompilerParams(
            dimension_semantics=("parallel", "parallel")
        ),
    )(wt, xs)
    return out.reshape(b, c, h, w)


# speedup vs baseline: 1.5911x; 1.0523x over previous
import jax
import jax.numpy as jnp
from jax.experimental import pallas as pl
from jax.experimental.pallas import tpu as pltpu

_C = 96
_EPS = 1e-06
_NORM = float(_C)
_LANES = 512
_CHUNK = 128


def _softmax_kernel(w_ref, wt_ref):
    w = w_ref[...]
    m = jnp.max(w)
    e = jnp.exp(w - m)
    wt_ref[...] = (e / jnp.sum(e)) * _NORM


def _rank_weight_kernel(wt_ref, x_ref, o_ref):
    c_iota = jax.lax.broadcasted_iota(jnp.int32, (_C, 1), 0)

    for s in range(_LANES // _CHUNK):
        x = x_ref[0, :, pl.ds(s * _CHUNK, _CHUNK)]

        def rank_body(j, acc):
            row = x_ref[0, pl.ds(j, 1), :][:, s * _CHUNK : (s + 1) * _CHUNK]
            beat = (row > x) | ((row >= x) & (c_iota > j))
            return acc + jnp.where(beat, jnp.int32(1), jnp.int32(0))

        rank = jax.lax.fori_loop(
            0, _C, rank_body, jnp.zeros(x.shape, jnp.int32), unroll=2
        )

        def sel_body(j, acc):
            w = wt_ref[j]
            return jnp.where(rank == j, w, acc)

        wt_g = jax.lax.fori_loop(
            0, _C, sel_body, jnp.zeros(x.shape, jnp.float32), unroll=2
        )
        o_ref[0, :, pl.ds(s * _CHUNK, _CHUNK)] = wt_g * jax.lax.rsqrt(
            x * x + _EPS
        )


@jax.jit
def kernel(x, weights):
    b, c, h, w = x.shape
    wt = pl.pallas_call(
        _softmax_kernel,
        out_shape=jax.ShapeDtypeStruct((1, _C), jnp.float32),
    )(weights.reshape(1, _C))
    wt = wt.reshape(_C)

    xs = x.reshape(b, c, h * w)
    n_s = (h * w) // _LANES
    out = pl.pallas_call(
        _rank_weight_kernel,
        grid=(b, n_s),
        in_specs=[
            pl.BlockSpec(memory_space=pltpu.SMEM),
            pl.BlockSpec((1, c, _LANES), lambda i, j: (i, 0, j)),
        ],
        out_specs=pl.BlockSpec((1, c, _LANES), lambda i, j: (i, 0, j)),
        out_shape=jax.ShapeDtypeStruct((b, c, h * w), jnp.float32),
        compiler_params=pltpu.CompilerParams(
            dimension_semantics=("parallel", "parallel")
        ),
    )(wt, xs)
    return out.reshape(b, c, h, w)


# speedup vs baseline: 2.0425x; 1.2837x over previous
import jax
import jax.numpy as jnp
from jax.experimental import pallas as pl
from jax.experimental.pallas import tpu as pltpu

_C = 96
_EPS = 1e-06
_NORM = float(_C)
_LANES = 512
_CHUNK = 128


def _softmax_kernel(w_ref, wt_ref):
    w = w_ref[...]
    m = jnp.max(w)
    e = jnp.exp(w - m)
    wt_ref[...] = (e / jnp.sum(e)) * _NORM


def _rank_weight_kernel(wtv_ref, x_ref, o_ref):
    c_iota = jax.lax.broadcasted_iota(jnp.int32, (_C, 1), 0)

    for s in range(_LANES // _CHUNK):
        x = x_ref[0, :, pl.ds(s * _CHUNK, _CHUNK)]

        def rank_body(j, acc):
            row = x_ref[0, pl.ds(j, 1), :][:, s * _CHUNK : (s + 1) * _CHUNK]
            beat = (row > x) | ((row >= x) & (c_iota > j))
            return acc + jnp.where(beat, jnp.int32(1), jnp.int32(0))

        rank = jax.lax.fori_loop(
            0, _C, rank_body, jnp.zeros(x.shape, jnp.int32), unroll=2
        )

        table = jnp.broadcast_to(wtv_ref[...], (_C, _C))
        wt_g = jnp.take_along_axis(table, rank, axis=1)
        o_ref[0, :, pl.ds(s * _CHUNK, _CHUNK)] = wt_g * jax.lax.rsqrt(
            x * x + _EPS
        )


@jax.jit
def kernel(x, weights):
    b, c, h, w = x.shape
    wt = pl.pallas_call(
        _softmax_kernel,
        out_shape=jax.ShapeDtypeStruct((1, _C), jnp.float32),
    )(weights.reshape(1, _C))

    xs = x.reshape(b, c, h * w)
    n_s = (h * w) // _LANES
    out = pl.pallas_call(
        _rank_weight_kernel,
        grid=(b, n_s),
        in_specs=[
            pl.BlockSpec((1, _C), lambda i, j: (0, 0)),
            pl.BlockSpec((1, c, _LANES), lambda i, j: (i, 0, j)),
        ],
        out_specs=pl.BlockSpec((1, c, _LANES), lambda i, j: (i, 0, j)),
        out_shape=jax.ShapeDtypeStruct((b, c, h * w), jnp.float32),
        compiler_params=pltpu.CompilerParams(
            dimension_semantics=("parallel", "parallel")
        ),
    )(wt, xs)
    return out.reshape(b, c, h, w)


# speedup vs baseline: 2.4488x; 1.1989x over previous
import jax
import jax.numpy as jnp
from jax.experimental import pallas as pl
from jax.experimental.pallas import tpu as pltpu

_C = 96
_EPS = 1e-06
_NORM = float(_C)
_LANES = 512
_CHUNK = 128


def _softmax_kernel(w_ref, wt_ref):
    w = w_ref[...]
    m = jnp.max(w)
    e = jnp.exp(w - m)
    wt_ref[...] = (e / jnp.sum(e)) * _NORM


def _rank_weight_kernel(wtv_ref, x_ref, o_ref):
    c_iota = jax.lax.broadcasted_iota(jnp.int32, (_C, 1), 0)

    for s in range(_LANES // _CHUNK):
        x = x_ref[0, :, pl.ds(s * _CHUNK, _CHUNK)]

        def rank_body(j, acc):
            row = x_ref[0, pl.ds(j, 1), :][:, s * _CHUNK : (s + 1) * _CHUNK]
            beat = (row > x) | ((row >= x) & (c_iota > j))
            return acc + jnp.where(beat, jnp.int32(1), jnp.int32(0))

        rank = jax.lax.fori_loop(
            0, _C, rank_body, jnp.zeros(x.shape, jnp.int32), unroll=8
        )

        table = jnp.broadcast_to(wtv_ref[...], (_C, _C))
        wt_g = jnp.take_along_axis(table, rank, axis=1)
        o_ref[0, :, pl.ds(s * _CHUNK, _CHUNK)] = wt_g * jax.lax.rsqrt(
            x * x + _EPS
        )


@jax.jit
def kernel(x, weights):
    b, c, h, w = x.shape
    wt = pl.pallas_call(
        _softmax_kernel,
        out_shape=jax.ShapeDtypeStruct((1, _C), jnp.float32),
    )(weights.reshape(1, _C))

    xs = x.reshape(b, c, h * w)
    n_s = (h * w) // _LANES
    out = pl.pallas_call(
        _rank_weight_kernel,
        grid=(b, n_s),
        in_specs=[
            pl.BlockSpec((1, _C), lambda i, j: (0, 0)),
            pl.BlockSpec((1, c, _LANES), lambda i, j: (i, 0, j)),
        ],
        out_specs=pl.BlockSpec((1, c, _LANES), lambda i, j: (i, 0, j)),
        out_shape=jax.ShapeDtypeStruct((b, c, h * w), jnp.float32),
        compiler_params=pltpu.CompilerParams(
            dimension_semantics=("parallel", "parallel")
        ),
    )(wt, xs)
    return out.reshape(b, c, h, w)


# speedup vs baseline: 4.9844x; 2.0355x over previous
import functools

import jax
import jax.numpy as jnp
from jax.experimental import pallas as pl
from jax.experimental.pallas import tpu as pltpu
from jax.experimental.pallas import tpu_sc as plsc

_C = 96
_EPS = 1e-06
_NORM = float(_C)
_LANES = 512
_CHUNK = 128
_G = 8
_NG = _C // _G

_HW = 224 * 224
_NB = 4
_SC_K = 4096
_S_TC = _HW - _SC_K
_SC_W = 32
_SC_PW = _SC_K // _SC_W
_NV = _SC_PW // 16


def _softmax_kernel(w_ref, wt_ref):
    w = w_ref[...]
    m = jnp.max(w)
    e = jnp.exp(w - m)
    wt_ref[...] = (e / jnp.sum(e)) * _NORM


def _rank_weight_kernel(wtv_ref, x_ref, o_ref):
    one = jnp.int32(1)
    zero = jnp.int32(0)
    giota = jax.lax.broadcasted_iota(jnp.int32, (_G, 1), 0)
    gmasks = [giota > r for r in range(_G)]
    for s in range(_LANES // _CHUNK):
        lo = s * _CHUNK
        x = x_ref[0, :, pl.ds(lo, _CHUNK)]
        xg = [x[g * _G : (g + 1) * _G, :] for g in range(_NG)]
        accg = [jnp.zeros((_G, _CHUNK), jnp.int32) for _ in range(_NG)]

        for j in range(_C):
            gj = j // _G
            row = x_ref[0, pl.ds(j, 1), pl.ds(lo, _CHUNK)]
            bmask = gmasks[j % _G]
            for g in range(_NG):
                if g < gj:
                    beat = row > xg[g]
                elif g > gj:
                    beat = row >= xg[g]
                else:
                    beat = (row > xg[g]) | ((row >= xg[g]) & bmask)
                accg[g] = accg[g] + jnp.where(beat, one, zero)

        rank = jnp.concatenate(accg, axis=0)
        table = jnp.broadcast_to(wtv_ref[...], (_C, _C))
        wt_g = jnp.take_along_axis(table, rank, axis=1)
        o_ref[0, :, pl.ds(lo, _CHUNK)] = wt_g * jax.lax.rsqrt(x * x + _EPS)


def _sc_body(wt_hbm, x_hbm, o_hbm, wt_v, x_v, o_v):
    wid = jax.lax.axis_index("s") * 2 + jax.lax.axis_index("c")
    base = _S_TC + wid * _SC_PW
    obase = wid * _SC_PW
    pltpu.sync_copy(wt_hbm, wt_v)

    wt_tabs = [wt_v[pl.ds(16 * t, 16)] for t in range(_C // 16)]

    def batch_body(b, carry):
        pltpu.sync_copy(x_hbm.at[b, :, pl.ds(base, _SC_PW)], x_v)
        for v in range(_NV):
            lo = v * 16

            def c_body(c, carry2):
                xc = x_v[c, pl.ds(lo, 16)]
                acc = jnp.zeros((16,), jnp.int32)
                for j in range(_C):
                    xj = x_v[j, pl.ds(lo, 16)]
                    acc = acc + jnp.where(xj > xc, 1, 0)

                def eq_body(j2, a):
                    xj2 = x_v[j2, pl.ds(lo, 16)]
                    return a + jnp.where(xj2 == xc, 1, 0)

                acc = jax.lax.fori_loop(0, c, eq_body, acc)
                hi = jax.lax.shift_right_logical(acc, 4)
                lo_i = jnp.bitwise_and(acc, 15)
                w = jnp.zeros((16,), jnp.float32)
                dn = jax.lax.GatherDimensionNumbers(
                    offset_dims=(),
                    collapsed_slice_dims=(0,),
                    start_index_map=(0,),
                )
                for t in range(_C // 16):
                    wt_t = jax.lax.gather(
                        wt_tabs[t],
                        lo_i[:, None],
                        dn,
                        (1,),
                        mode=jax.lax.GatherScatterMode.PROMISE_IN_BOUNDS,
                    )
                    w = jnp.where(hi == t, wt_t, w)
                y = xc * xc + _EPS
                i = jax.lax.bitcast_convert_type(y, jnp.int32)
                i = jnp.int32(0x5F3759DF) - jax.lax.shift_right_logical(i, 1)
                r = jax.lax.bitcast_convert_type(i, jnp.float32)
                h = 0.5 * y
                for _ in range(3):
                    r = r * (1.5 - h * r * r)
                o_v[c, pl.ds(lo, 16)] = w * r
                return carry2

            jax.lax.fori_loop(0, _C, c_body, 0)
        pltpu.sync_copy(o_v, o_hbm.at[b, :, pl.ds(obase, _SC_PW)])
        return carry

    jax.lax.fori_loop(0, _NB, batch_body, 0)


_sc_call = functools.partial(
    pl.kernel,
    out_type=jax.ShapeDtypeStruct((_NB, _C, _SC_K), jnp.float32),
    mesh=plsc.VectorSubcoreMesh(core_axis_name="c", subcore_axis_name="s"),
    scratch_types=[
        pltpu.VMEM((_C,), jnp.float32),
        pltpu.VMEM((_C, _SC_PW), jnp.float32),
        pltpu.VMEM((_C, _SC_PW), jnp.float32),
    ],
)(_sc_body)


@jax.jit
def kernel(x, weights):
    b, c, h, w = x.shape
    wt = pl.pallas_call(
        _softmax_kernel,
        out_shape=jax.ShapeDtypeStruct((1, _C), jnp.float32),
    )(weights.reshape(1, _C))

    xs = x.reshape(b, c, h * w)
    n_tc = _S_TC // _LANES
    out_tc = pl.pallas_call(
        _rank_weight_kernel,
        grid=(b, n_tc),
        in_specs=[
            pl.BlockSpec((1, _C), lambda i, j: (0, 0)),
            pl.BlockSpec((1, c, _LANES), lambda i, j: (i, 0, j)),
        ],
        out_specs=pl.BlockSpec((1, c, _LANES), lambda i, j: (i, 0, j)),
        out_shape=jax.ShapeDtypeStruct((b, c, _S_TC), jnp.float32),
        compiler_params=pltpu.CompilerParams(
            dimension_semantics=("parallel", "parallel")
        ),
    )(wt, xs)

    out_sc = _sc_call(wt.reshape(_C), xs)
    out = jnp.concatenate([out_tc, out_sc], axis=2)
    return out.reshape(b, c, h, w)


# speedup vs baseline: 6.1053x; 1.2249x over previous
import functools

import jax
import jax.numpy as jnp
from jax.experimental import pallas as pl
from jax.experimental.pallas import tpu as pltpu
from jax.experimental.pallas import tpu_sc as plsc

_C = 96
_EPS = 1e-06
_NORM = float(_C)
_LANES = 512
_CHUNK = 128
_G = 8
_NG = _C // _G

_HW = 224 * 224
_NB = 4
_SC_K = 3072
_S_TC = _HW - _SC_K
_SC_W = 32
_SC_CHUNK = 128
_SC_NCHUNK = _NB * _SC_K // _SC_CHUNK
_SC_PW = _SC_NCHUNK // _SC_W
_CPB = _SC_K // _SC_CHUNK
_NV = _SC_CHUNK // 16


def _softmax_kernel(w_ref, wt_ref):
    w = w_ref[...]
    m = jnp.max(w)
    e = jnp.exp(w - m)
    wt_ref[...] = (e / jnp.sum(e)) * _NORM


def _rank_weight_kernel(wtv_ref, x_ref, o_ref):
    one = jnp.int32(1)
    zero = jnp.int32(0)
    giota = jax.lax.broadcasted_iota(jnp.int32, (_G, 1), 0)
    gmasks = [giota > r for r in range(_G)]
    for s in range(_LANES // _CHUNK):
        lo = s * _CHUNK
        x = x_ref[0, :, pl.ds(lo, _CHUNK)]
        xg = [x[g * _G : (g + 1) * _G, :] for g in range(_NG)]
        accg = [jnp.zeros((_G, _CHUNK), jnp.int32) for _ in range(_NG)]

        for j in range(_C):
            gj = j // _G
            row = x_ref[0, pl.ds(j, 1), pl.ds(lo, _CHUNK)]
            bmask = gmasks[j % _G]
            for g in range(_NG):
                if g < gj:
                    beat = row > xg[g]
                elif g > gj:
                    beat = row >= xg[g]
                else:
                    beat = (row > xg[g]) | ((row >= xg[g]) & bmask)
                accg[g] = accg[g] + jnp.where(beat, one, zero)

        rank = jnp.concatenate(accg, axis=0)
        table = jnp.broadcast_to(wtv_ref[...], (_C, _C))
        wt_g = jnp.take_along_axis(table, rank, axis=1)
        o_ref[0, :, pl.ds(lo, _CHUNK)] = wt_g * jax.lax.rsqrt(x * x + _EPS)


def _sc_body(wt_hbm, x_hbm, o_hbm, wt_v, x_v, o_v):
    wid = jax.lax.axis_index("s") * 2 + jax.lax.axis_index("c")
    pltpu.sync_copy(wt_hbm, wt_v)

    wt_tabs = [wt_v[pl.ds(16 * t, 16)] for t in range(_C // 16)]

    def chunk_body(k, carry):
        chunk = wid * _SC_PW + k
        b = chunk // _CPB
        off = (chunk % _CPB) * _SC_CHUNK
        pltpu.sync_copy(x_hbm.at[b, :, pl.ds(_S_TC + off, _SC_CHUNK)], x_v)
        for v in range(_NV):
            lo = v * 16

            def c_body(c, carry2):
                xc = x_v[c, pl.ds(lo, 16)]
                acc = jnp.zeros((16,), jnp.int32)
                for j in range(_C):
                    xj = x_v[j, pl.ds(lo, 16)]
                    acc = acc + jnp.where(xj > xc, 1, 0)

                def eq_body(j2, a):
                    xj2 = x_v[j2, pl.ds(lo, 16)]
                    return a + jnp.where(xj2 == xc, 1, 0)

                acc = jax.lax.fori_loop(0, c, eq_body, acc)
                hi = jax.lax.shift_right_logical(acc, 4)
                lo_i = jnp.bitwise_and(acc, 15)
                w = jnp.zeros((16,), jnp.float32)
                dn = jax.lax.GatherDimensionNumbers(
                    offset_dims=(),
                    collapsed_slice_dims=(0,),
                    start_index_map=(0,),
                )
                for t in range(_C // 16):
                    wt_t = jax.lax.gather(
                        wt_tabs[t],
                        lo_i[:, None],
                        dn,
                        (1,),
                        mode=jax.lax.GatherScatterMode.PROMISE_IN_BOUNDS,
                    )
                    w = jnp.where(hi == t, wt_t, w)
                y = xc * xc + _EPS
                i = jax.lax.bitcast_convert_type(y, jnp.int32)
                i = jnp.int32(0x5F3759DF) - jax.lax.shift_right_logical(i, 1)
                r = jax.lax.bitcast_convert_type(i, jnp.float32)
                h = 0.5 * y
                for _ in range(3):
                    r = r * (1.5 - h * r * r)
                o_v[c, pl.ds(lo, 16)] = w * r
                return carry2

            jax.lax.fori_loop(0, _C, c_body, 0)
        pltpu.sync_copy(o_v, o_hbm.at[b, :, pl.ds(off, _SC_CHUNK)])
        return carry

    jax.lax.fori_loop(0, _SC_PW, chunk_body, 0)


_sc_call = functools.partial(
    pl.kernel,
    out_type=jax.ShapeDtypeStruct((_NB, _C, _SC_K), jnp.float32),
    mesh=plsc.VectorSubcoreMesh(core_axis_name="c", subcore_axis_name="s"),
    scratch_types=[
        pltpu.VMEM((_C,), jnp.float32),
        pltpu.VMEM((_C, _SC_CHUNK), jnp.float32),
        pltpu.VMEM((_C, _SC_CHUNK), jnp.float32),
    ],
)(_sc_body)


@jax.jit
def kernel(x, weights):
    b, c, h, w = x.shape
    wt = pl.pallas_call(
        _softmax_kernel,
        out_shape=jax.ShapeDtypeStruct((1, _C), jnp.float32),
    )(weights.reshape(1, _C))

    xs = x.reshape(b, c, h * w)
    n_tc = _S_TC // _LANES
    out_tc = pl.pallas_call(
        _rank_weight_kernel,
        grid=(b, n_tc),
        in_specs=[
            pl.BlockSpec((1, _C), lambda i, j: (0, 0)),
            pl.BlockSpec((1, c, _LANES), lambda i, j: (i, 0, j)),
        ],
        out_specs=pl.BlockSpec((1, c, _LANES), lambda i, j: (i, 0, j)),
        out_shape=jax.ShapeDtypeStruct((b, c, h * w), jnp.float32),
        compiler_params=pltpu.CompilerParams(
            dimension_semantics=("parallel", "parallel")
        ),
    )(wt, xs)

    out_sc = _sc_call(wt.reshape(_C), xs)
    out = jax.lax.dynamic_update_slice(out_tc, out_sc, (0, 0, _S_TC))
    return out.reshape(b, c, h, w)
